# baseline (speedup 1.0000x reference)
import jax
import jax.numpy as jnp
from jax.experimental import pallas as pl
from jax.experimental.pallas import tpu as pltpu

B = 4096
D = 128
BB = 128


def _body(g_ref, p_ref, h_ref, pg_ref, pp_ref, wx_ref, wy_ref,
          og_ref, op_ref, oh_ref):
    og_ref[...] = g_ref[...] + pg_ref[...]
    op_ref[...] = p_ref[...] + pp_ref[...]
    oh_ref[...] = h_ref[...] + wy_ref[...][:, None, :] + wx_ref[...]


def kernel(global_proj, player_proj, hex_proj, pos_global, pos_player, W_x, W_y):
    hex4 = hex_proj.reshape(B, 11, 15, D)
    grid = (B // BB,)
    out = pl.pallas_call(
        _body,
        grid=grid,
        in_specs=[
            pl.BlockSpec((BB, 1, D), lambda i: (i, 0, 0)),
            pl.BlockSpec((BB, 2, D), lambda i: (i, 0, 0)),
            pl.BlockSpec((BB, 11, 15, D), lambda i: (i, 0, 0, 0)),
            pl.BlockSpec((1, D), lambda i: (0, 0)),
            pl.BlockSpec((2, D), lambda i: (0, 0)),
            pl.BlockSpec((15, D), lambda i: (0, 0)),
            pl.BlockSpec((11, D), lambda i: (0, 0)),
        ],
        out_specs=[
            pl.BlockSpec((BB, 1, D), lambda i: (i, 0, 0)),
            pl.BlockSpec((BB, 2, D), lambda i: (i, 0, 0)),
            pl.BlockSpec((BB, 11, 15, D), lambda i: (i, 0, 0, 0)),
        ],
        out_shape=[
            jax.ShapeDtypeStruct((B, 1, D), jnp.float32),
            jax.ShapeDtypeStruct((B, 2, D), jnp.float32),
            jax.ShapeDtypeStruct((B, 11, 15, D), jnp.float32),
        ],
    )(global_proj, player_proj, hex4, pos_global, pos_player, W_x, W_y)
    g, p, h4 = out
    return (g, p, h4.reshape(B, 165, D))


# speedup vs baseline: 1.0227x; 1.0227x over previous
import jax
import jax.numpy as jnp
from jax.experimental import pallas as pl
from jax.experimental.pallas import tpu as pltpu

B = 4096
D = 128
BB = 128


def _body(g_ref, p_ref, h_ref, pg_ref, pp_ref, wx_ref, wy_ref,
          og_ref, op_ref, oh_ref, pe_ref):
    @pl.when(pl.program_id(0) == 0)
    def _fill_pe():
        wx = wx_ref[...]
        for y in range(11):
            pe_ref[pl.ds(15 * y, 15), :] = wy_ref[y:y + 1, :] + wx

    og_ref[...] = g_ref[...] + pg_ref[...]
    op_ref[...] = p_ref[...] + pp_ref[...]
    oh_ref[...] = h_ref[...] + pe_ref[...]


def kernel(global_proj, player_proj, hex_proj, pos_global, pos_player, W_x, W_y):
    grid = (B // BB,)
    out = pl.pallas_call(
        _body,
        grid=grid,
        in_specs=[
            pl.BlockSpec((BB, 1, D), lambda i: (i, 0, 0)),
            pl.BlockSpec((BB, 2, D), lambda i: (i, 0, 0)),
            pl.BlockSpec((BB, 165, D), lambda i: (i, 0, 0)),
            pl.BlockSpec((1, D), lambda i: (0, 0)),
            pl.BlockSpec((2, D), lambda i: (0, 0)),
            pl.BlockSpec((15, D), lambda i: (0, 0)),
            pl.BlockSpec((11, D), lambda i: (0, 0)),
        ],
        out_specs=[
            pl.BlockSpec((BB, 1, D), lambda i: (i, 0, 0)),
            pl.BlockSpec((BB, 2, D), lambda i: (i, 0, 0)),
            pl.BlockSpec((BB, 165, D), lambda i: (i, 0, 0)),
        ],
        out_shape=[
            jax.ShapeDtypeStruct((B, 1, D), jnp.float32),
            jax.ShapeDtypeStruct((B, 2, D), jnp.float32),
            jax.ShapeDtypeStruct((B, 165, D), jnp.float32),
        ],
        scratch_shapes=[pltpu.VMEM((165, D), jnp.float32)],
    )(global_proj, player_proj, hex_proj, pos_global, pos_player, W_x, W_y)
    return tuple(out)
